# baseline (speedup 1.0000x reference)
import functools

import jax
import jax.numpy as jnp
from jax import lax
from jax.experimental import pallas as pl
from jax.experimental.pallas import tpu as pltpu

ROWS = 16384
COLS = 1024
BLK = 256
NBLK = ROWS // BLK


def _tc_body(x_ref, o_ref):
    o_ref[0, 0, :] = jnp.argmax(x_ref[...], axis=-1).astype(jnp.int32)


_argmax_tc = pl.pallas_call(
    _tc_body,
    grid=(NBLK,),
    in_specs=[pl.BlockSpec((BLK, COLS), lambda i: (i, 0))],
    out_specs=pl.BlockSpec((1, 1, BLK), lambda i: (i, 0, 0)),
    out_shape=jax.ShapeDtypeStruct((NBLK, 1, BLK), jnp.int32),
)


def kernel(emission, to_string):
    del to_string
    return _argmax_tc(emission).reshape(ROWS)


# speedup vs baseline: 1.0927x; 1.0927x over previous
import functools

import jax
import jax.numpy as jnp
from jax import lax
from jax.experimental import pallas as pl
from jax.experimental.pallas import tpu as pltpu

ROWS = 16384
COLS = 1024
BLK = 256
NBLK = ROWS // BLK


def _tc_body(x_ref, o_ref):
    x = x_ref[...]
    m = jnp.max(x, axis=-1, keepdims=True)
    ii = lax.broadcasted_iota(jnp.int32, (BLK, COLS), 1)
    cand = jnp.where(x == m, ii, COLS)
    o_ref[0, 0, :] = jnp.min(cand, axis=-1)


_argmax_tc = pl.pallas_call(
    _tc_body,
    grid=(NBLK,),
    in_specs=[pl.BlockSpec((BLK, COLS), lambda i: (i, 0))],
    out_specs=pl.BlockSpec((1, 1, BLK), lambda i: (i, 0, 0)),
    out_shape=jax.ShapeDtypeStruct((NBLK, 1, BLK), jnp.int32),
)


def kernel(emission, to_string):
    del to_string
    return _argmax_tc(emission).reshape(ROWS)
